# baseline (speedup 1.0000x reference)
import functools

import jax
import jax.numpy as jnp
import numpy as np
from jax import lax
from jax.experimental import pallas as pl
from jax.experimental.pallas import tpu as pltpu
from jax.experimental.pallas import tpu_sc as plsc

F32 = jnp.float32
NEG_BIG = -1e30


_NC = 2
_NS = 16
_NW = _NC * _NS


def _sc_mesh():
    return plsc.VectorSubcoreMesh(core_axis_name="c", subcore_axis_name="s")


_K = 6


@functools.lru_cache(maxsize=None)
def _make_sc_gather(n_idx, n_rows, d, dtype_name, n_lists):
    dt = jnp.dtype(dtype_name)
    per_w = n_idx // _NW
    ch = min(128, per_w)
    n_ch = per_w // ch
    rem = per_w - n_ch * ch
    k = max(1, min(_K, n_ch))
    groups = n_ch // k
    tail = n_ch - groups * k

    scratch = [
        pltpu.VMEM((max(per_w, 8),), jnp.int32),
        pltpu.VMEM((k, ch, d), dt),
    ] + [pltpu.SemaphoreType.DMA] * (k + 1)

    @functools.partial(
        pl.kernel,
        out_type=jax.ShapeDtypeStruct((n_lists * n_idx, d), dt),
        mesh=_sc_mesh(),
        scratch_types=scratch,
    )
    def gather_k(table_hbm, *args):
        idx_hbms = args[:n_lists]
        out_hbm = args[n_lists]
        idx_v, rows_v = args[n_lists + 1:n_lists + 3]
        sems = args[n_lists + 3:]
        gsems, osem = sems[:k], sems[k]
        wid = lax.axis_index("s") * _NC + lax.axis_index("c")
        base = wid * per_w

        def run(idx_hbm, out_base):
            pltpu.sync_copy(idx_hbm.at[pl.ds(base, per_w)],
                            idx_v.at[pl.ds(0, per_w)])

            def group_body(g, carry):
                off = g * (k * ch)
                gd = [
                    pltpu.async_copy(
                        table_hbm.at[idx_v.at[pl.ds(off + b * ch, ch)]],
                        rows_v.at[b], gsems[b])
                    for b in range(k)
                ]
                od = []
                for b in range(k):
                    gd[b].wait()
                    od.append(pltpu.async_copy(
                        rows_v.at[b],
                        out_hbm.at[pl.ds(out_base + off + b * ch, ch)], osem))
                for dsc in od:
                    dsc.wait()
                return carry

            if groups:
                lax.fori_loop(0, groups, group_body, 0)
            for j in range(tail):
                off = (groups * k + j) * ch
                pltpu.async_copy(
                    table_hbm.at[idx_v.at[pl.ds(off, ch)]],
                    rows_v.at[0], gsems[0]).wait()
                pltpu.sync_copy(rows_v.at[0],
                                out_hbm.at[pl.ds(out_base + off, ch)])
            if rem:
                off = n_ch * ch
                pltpu.async_copy(
                    table_hbm.at[idx_v.at[pl.ds(off, rem)]],
                    rows_v.at[0].at[pl.ds(0, rem)], gsems[0]).wait()
                pltpu.sync_copy(rows_v.at[0].at[pl.ds(0, rem)],
                                out_hbm.at[pl.ds(out_base + off, rem)])

        for j, idx_hbm in enumerate(idx_hbms):
            run(idx_hbm, j * n_idx + base)

    return gather_k


def _pad_rows(n_rows):
    unit = 8 * _NS
    return ((n_rows + unit - 1) // unit) * unit


@functools.lru_cache(maxsize=None)
def _make_sc_scatter(n_idx, n_rows, d, with_count):
    per_w = n_idx // _NW
    ch = min(64, per_w)
    n_ch = per_w // ch
    rem = per_w - n_ch * ch
    k = max(1, min(3 if with_count else 5, n_ch))
    groups = n_ch // k
    tail = n_ch - groups * k
    n_pad = _pad_rows(n_rows)
    rows_per_tile = n_pad // _NS

    scratch = (
        [pltpu.VMEM((ch,), jnp.int32)] * k
        + [pltpu.VMEM((ch, d), F32)] * k
        + [pltpu.VMEM_SHARED((n_pad, d), F32)]
        + [pltpu.SemaphoreType.DMA] * (2 * k + 1)
    )
    out_type = [jax.ShapeDtypeStruct((2 * n_pad, d), F32)]
    if with_count:
        scratch = scratch + [pltpu.VMEM((ch, 16), F32),
                             pltpu.VMEM_SHARED((n_pad, 16), F32)]
        out_type = out_type + [jax.ShapeDtypeStruct((2 * n_pad, 16), F32)]

    @functools.partial(
        pl.kernel,
        out_type=out_type,
        mesh=_sc_mesh(),
        scratch_types=scratch,
    )
    def scatter_k(*args):
        if with_count:
            (vals_hbm, idx_hbm, zeros_hbm, ones_hbm, zeros16_hbm,
             out_hbm, cnt_hbm) = args[:7]
            rest = args[7:]
        else:
            vals_hbm, idx_hbm, zeros_hbm, out_hbm = args[:4]
            rest = args[4:]
        idx_vs = rest[:k]
        rows_vs = rest[k:2 * k]
        acc_sh = rest[2 * k]
        isems = rest[2 * k + 1:3 * k + 1]
        msems = rest[3 * k + 1:4 * k + 1]
        ssem = rest[4 * k + 1]
        if with_count:
            ones_v, acc16_sh = rest[4 * k + 2:4 * k + 4]
        c = lax.axis_index("c")
        s = lax.axis_index("s")
        wid = s * _NC + c
        base = wid * per_w
        pltpu.sync_copy(
            zeros_hbm.at[pl.ds(s * rows_per_tile, rows_per_tile)],
            acc_sh.at[pl.ds(s * rows_per_tile, rows_per_tile)],
        )
        if with_count:
            pltpu.sync_copy(ones_hbm, ones_v)
            pltpu.sync_copy(
                zeros16_hbm.at[pl.ds(s * rows_per_tile, rows_per_tile)],
                acc16_sh.at[pl.ds(s * rows_per_tile, rows_per_tile)],
            )
        plsc.subcore_barrier()

        def group_body(g, carry):
            off = base + g * (k * ch)
            idd = [pltpu.async_copy(idx_hbm.at[pl.ds(off + b * ch, ch)],
                                    idx_vs[b], isems[b]) for b in range(k)]
            mdd = [pltpu.async_copy(vals_hbm.at[pl.ds(off + b * ch, ch)],
                                    rows_vs[b], msems[b]) for b in range(k)]
            sd = []
            for b in range(k):
                idd[b].wait()
                mdd[b].wait()
                sd.append(pltpu.async_copy(
                    rows_vs[b], acc_sh.at[idx_vs[b]], ssem, add=True))
                if with_count:
                    sd.append(pltpu.async_copy(
                        ones_v, acc16_sh.at[idx_vs[b]], ssem, add=True))
            for dsc in sd:
                dsc.wait()
            return carry

        if groups:
            lax.fori_loop(0, groups, group_body, 0)

        def do_serial(off, cnt):
            pltpu.sync_copy(idx_hbm.at[pl.ds(off, cnt)],
                            idx_vs[0].at[pl.ds(0, cnt)])
            pltpu.sync_copy(vals_hbm.at[pl.ds(off, cnt)],
                            rows_vs[0].at[pl.ds(0, cnt)])
            pltpu.sync_copy(
                rows_vs[0].at[pl.ds(0, cnt)],
                acc_sh.at[idx_vs[0].at[pl.ds(0, cnt)]],
                add=True,
            )
            if with_count:
                pltpu.sync_copy(
                    ones_v.at[pl.ds(0, cnt)],
                    acc16_sh.at[idx_vs[0].at[pl.ds(0, cnt)]],
                    add=True,
                )

        for j in range(tail):
            do_serial(base + (groups * k + j) * ch, ch)
        if rem:
            do_serial(base + n_ch * ch, rem)
        plsc.subcore_barrier()
        pltpu.sync_copy(
            acc_sh.at[pl.ds(s * rows_per_tile, rows_per_tile)],
            out_hbm.at[pl.ds(c * n_pad + s * rows_per_tile, rows_per_tile)],
        )
        if with_count:
            pltpu.sync_copy(
                acc16_sh.at[pl.ds(s * rows_per_tile, rows_per_tile)],
                cnt_hbm.at[pl.ds(c * n_pad + s * rows_per_tile, rows_per_tile)],
            )

    return scatter_k


@functools.lru_cache(maxsize=None)
def _make_sc_count(n_idx, n_rows, d):
    per_w = n_idx // _NW
    ch = min(128, per_w)
    n_ch = per_w // ch
    rem = per_w - n_ch * ch
    k = max(1, min(_K, n_ch))
    groups = n_ch // k
    tail = n_ch - groups * k
    n_pad = _pad_rows(n_rows)
    rows_per_tile = n_pad // _NS

    scratch = (
        [pltpu.VMEM((ch,), jnp.int32)] * k
        + [pltpu.VMEM((ch, d), F32)]
        + [pltpu.VMEM_SHARED((n_pad, d), F32)]
        + [pltpu.SemaphoreType.DMA] * (k + 1)
    )

    @functools.partial(
        pl.kernel,
        out_type=jax.ShapeDtypeStruct((2 * n_pad, d), F32),
        mesh=_sc_mesh(),
        scratch_types=scratch,
    )
    def count_k(idx_hbm, ones_hbm, zeros_hbm, out_hbm, *rest):
        idx_vs = rest[:k]
        ones_v = rest[k]
        acc_sh = rest[k + 1]
        isems = rest[k + 2:2 * k + 2]
        ssem = rest[2 * k + 2]
        c = lax.axis_index("c")
        s = lax.axis_index("s")
        wid = s * _NC + c
        base = wid * per_w
        pltpu.sync_copy(ones_hbm, ones_v)
        pltpu.sync_copy(
            zeros_hbm.at[pl.ds(s * rows_per_tile, rows_per_tile)],
            acc_sh.at[pl.ds(s * rows_per_tile, rows_per_tile)],
        )
        plsc.subcore_barrier()

        def group_body(g, carry):
            off = base + g * (k * ch)
            idd = [pltpu.async_copy(idx_hbm.at[pl.ds(off + b * ch, ch)],
                                    idx_vs[b], isems[b]) for b in range(k)]
            sd = []
            for b in range(k):
                idd[b].wait()
                sd.append(pltpu.async_copy(
                    ones_v, acc_sh.at[idx_vs[b]], ssem, add=True))
            for dsc in sd:
                dsc.wait()
            return carry

        if groups:
            lax.fori_loop(0, groups, group_body, 0)

        def do_serial(off, cnt):
            pltpu.sync_copy(idx_hbm.at[pl.ds(off, cnt)],
                            idx_vs[0].at[pl.ds(0, cnt)])
            pltpu.sync_copy(
                ones_v.at[pl.ds(0, cnt)],
                acc_sh.at[idx_vs[0].at[pl.ds(0, cnt)]],
                add=True,
            )

        for j in range(tail):
            do_serial(base + (groups * k + j) * ch, ch)
        if rem:
            do_serial(base + n_ch * ch, rem)
        plsc.subcore_barrier()
        pltpu.sync_copy(
            acc_sh.at[pl.ds(s * rows_per_tile, rows_per_tile)],
            out_hbm.at[pl.ds(c * n_pad + s * rows_per_tile, rows_per_tile)],
        )

    return count_k


def _count_rows(idx, n_rows, ones, zeros):
    return _make_sc_count(idx.shape[0], n_rows, ones.shape[1])(
        idx, ones, zeros)


def _gather_rows(table, idx):
    return _make_sc_gather(idx.shape[0], table.shape[0], table.shape[1],
                           str(table.dtype), 1)(table, idx)


def _gather_rows2(table, idx_a, idx_b):
    return _make_sc_gather(idx_a.shape[0], table.shape[0], table.shape[1],
                           str(table.dtype), 2)(table, idx_a, idx_b)


def _scatter_rows(vals, idx, n_rows, zeros):
    out = _make_sc_scatter(vals.shape[0], n_rows, vals.shape[1], False)(
        vals, idx, zeros)
    return out[0]


def _scatter_rows_count(vals, idx, n_rows, zeros, ones16, zeros16):
    return _make_sc_scatter(vals.shape[0], n_rows, vals.shape[1], True)(
        vals, idx, zeros, ones16, zeros16)


def _gelu(v):
    return jax.nn.gelu(v)


def _full_spec(shape):
    nd = len(shape)
    return pl.BlockSpec(shape, lambda i, _nd=nd: (0,) * _nd)


def _proj_kernel(x_ref, w_ref, b_ref, o_ref):
    o_ref[...] = jnp.dot(x_ref[...], w_ref[...]) + b_ref[...]


def _linproj(xin, wt, b, blk):
    n, din = xin.shape
    dout = wt.shape[1]
    grid = n // blk
    return pl.pallas_call(
        _proj_kernel,
        grid=(grid,),
        in_specs=[
            pl.BlockSpec((blk, din), lambda i: (i, 0)),
            _full_spec((din, dout)),
            _full_spec((1, dout)),
        ],
        out_specs=pl.BlockSpec((blk, dout), lambda i: (i, 0)),
        out_shape=jax.ShapeDtypeStruct((n, dout), F32),
    )(xin, wt, b)


def _prelude_kernel(qa_ref, g1_ref, gb1_ref, g2_ref, gb2_ref, g3_ref, gb3_ref,
                    w1ue_ref, w1uw_ref, w1uu_ref, qw_ref, qb_ref,
                    u_ref, u1_ref, u2_ref, u3_ref, qs_ref):
    qa = qa_ref[...]
    h = _gelu(jnp.dot(qa, g1_ref[...]) + gb1_ref[...])
    h = _gelu(jnp.dot(h, g2_ref[...]) + gb2_ref[...])
    u = jnp.dot(h, g3_ref[...]) + gb3_ref[...]
    u_ref[...] = u
    u1_ref[...] = jnp.dot(u, w1ue_ref[...])
    u2_ref[...] = jnp.dot(u, w1uw_ref[...])
    u3_ref[...] = jnp.dot(u, w1uu_ref[...])
    qs_ref[...] = jnp.dot(qa, qw_ref[...]) + qb_ref[...]


def _prelude(qa, gw, w1ue, w1uw, w1uu, qw, qb):
    bsz, dfeat = qa.shape
    hid = gw[0][0].shape[1]
    outs = [jax.ShapeDtypeStruct((bsz, hid), F32)] * 5
    specs = [_full_spec((bsz, dfeat))]
    for (wt, b) in gw:
        specs.append(_full_spec(wt.shape))
        specs.append(_full_spec(b.shape))
    specs += [_full_spec(w1ue.shape), _full_spec(w1uw.shape),
              _full_spec(w1uu.shape), _full_spec(qw.shape), _full_spec(qb.shape)]
    return pl.pallas_call(
        _prelude_kernel,
        grid=(1,),
        in_specs=specs,
        out_specs=[_full_spec((bsz, hid))] * 5,
        out_shape=outs,
    )(qa, gw[0][0], gw[0][1], gw[1][0], gw[1][1], gw[2][0], gw[2][1],
      w1ue, w1uw, w1uu, qw, qb)


def _onehot_from_bounds(rowf, tlo, thi):
    ge = rowf >= tlo
    lt = rowf < thi
    return jnp.where(ge & lt, 1.0, 0.0).astype(F32)


def _wts_and_max(ea, oh, pid, u2_ref, w1_ref, b1_ref, w2_ref, b2_ref,
                 wts_ref, m_ref):
    @pl.when(pid == 0)
    def _():
        m_ref[...] = jnp.full(m_ref.shape, NEG_BIG, F32)

    h = _gelu(jnp.dot(ea, w1_ref[...]) + jnp.dot(oh, u2_ref[...])
              + b1_ref[...])
    wts = jnp.dot(h, w2_ref[...]) + b2_ref[...]
    wts_ref[...] = wts
    contrib = jnp.where(oh > 0, wts, NEG_BIG)
    m_ref[...] = jnp.maximum(m_ref[...],
                             jnp.max(contrib, axis=0, keepdims=True))


def _ea_wts_kernel(ea16_ref, wr_ref, wrb_ref, rowf_ref, tlo_ref, thi_ref,
                   u2_ref, w1_ref, b1_ref, w2_ref, b2_ref,
                   ea_ref, wts_ref, m_ref):
    pid = pl.program_id(0)
    ea = jnp.dot(ea16_ref[...], wr_ref[...]) + wrb_ref[...]
    ea_ref[...] = ea
    oh = _onehot_from_bounds(rowf_ref[...], tlo_ref[...], thi_ref[...])
    _wts_and_max(ea, oh, pid, u2_ref, w1_ref, b1_ref, w2_ref, b2_ref,
                 wts_ref, m_ref)


def _segsum_kernel(nblk, wts_ref, rowf_ref, tlo_ref, thi_ref, m_ref, s_ref):
    pid = pl.program_id(0)

    @pl.when(pid == 0)
    def _():
        s_ref[...] = jnp.zeros(s_ref.shape, F32)

    oh = _onehot_from_bounds(rowf_ref[...], tlo_ref[...], thi_ref[...])
    m_e = jnp.sum(oh * m_ref[...], axis=1, keepdims=True)
    e = jnp.exp(wts_ref[...] - m_e)
    s_ref[...] = s_ref[...] + jnp.sum(oh * e, axis=0, keepdims=True)


def _edge_kernel(nblk, is_last,
                 src_ref, dest_ref, ea_ref, rowf_ref, wts_ref,
                 tlo_ref, thi_ref, m_ref, s_ref, u1_ref,
                 w1s_ref, w1d_ref, w1e_ref, b1_ref, w2_ref, b2_ref,
                 w3_ref, b3_ref,
                 m1x_ref, m1e_ref, mb1_ref, m2_ref, mb2_ref, m3_ref, mb3_ref,
                 *refs):
    if is_last:
        ne_ref, msgs_ref, sig_ref, norm_ref, pooled_ref = refs
    else:
        (u2_ref, ww1_ref, wwb1_ref, ww2_ref, wwb2_ref,
         ne_ref, msgs_ref, wtsn_ref, mn_ref) = refs
    pid = pl.program_id(0)
    oh = _onehot_from_bounds(rowf_ref[...], tlo_ref[...], thi_ref[...])
    wts = wts_ref[...]
    m_e = jnp.sum(oh * m_ref[...], axis=1, keepdims=True)
    s_e = jnp.sum(oh * s_ref[...], axis=1, keepdims=True)
    norm = jnp.exp(wts - m_e) / jnp.maximum(s_e, 1e-16)
    src_b = src_ref[...].astype(jnp.bfloat16)
    dest_b = dest_ref[...].astype(jnp.bfloat16)
    h = _gelu(jnp.dot(src_b, w1s_ref[...], preferred_element_type=F32)
              + jnp.dot(dest_b, w1d_ref[...], preferred_element_type=F32)
              + jnp.dot(ea_ref[...], w1e_ref[...]) + jnp.dot(oh, u1_ref[...])
              + b1_ref[...])
    h = _gelu(jnp.dot(h, w2_ref[...]) + b2_ref[...])
    ne = jnp.dot(h, w3_ref[...]) + b3_ref[...]
    ne_ref[...] = ne
    g = _gelu(jnp.dot(src_b, m1x_ref[...], preferred_element_type=F32)
              + jnp.dot(ne, m1e_ref[...]) + mb1_ref[...])
    g = _gelu(jnp.dot(g, m2_ref[...]) + mb2_ref[...])
    msg = jnp.dot(g, m3_ref[...]) + mb3_ref[...]
    msgs_ref[...] = msg * norm
    if is_last:
        sig_ref[...] = jax.nn.sigmoid(wts)
        norm_ref[...] = norm

        @pl.when(pid == 0)
        def _():
            pooled_ref[...] = jnp.zeros(pooled_ref.shape, F32)

        pooled_ref[...] = pooled_ref[...] + lax.dot_general(
            oh, ne * norm, (((0,), (0,)), ((), ())))
    else:
        _wts_and_max(ne, oh, pid, u2_ref, ww1_ref, wwb1_ref, ww2_ref,
                     wwb2_ref, wtsn_ref, mn_ref)


def _node_kernel(xold_ref, p0_ref, p1_ref, c0_ref, c1_ref, nbf_ref, u3_ref,
                 w1x_ref, w1r_ref, b1_ref, w2_ref, b2_ref, w3_ref, b3_ref,
                 xnew_ref):
    cnt = c0_ref[...][:, :1] + c1_ref[...][:, :1]
    received = (p0_ref[...] + p1_ref[...]) / jnp.maximum(cnt, 1.0)
    seg = lax.broadcasted_iota(jnp.int32, (1, 64), 1).astype(F32)
    ohn = jnp.where(nbf_ref[...] == seg, 1.0, 0.0).astype(F32)
    h = _gelu(jnp.dot(xold_ref[...], w1x_ref[...])
              + jnp.dot(received, w1r_ref[...])
              + jnp.dot(ohn, u3_ref[...]) + b1_ref[...])
    h = _gelu(jnp.dot(h, w2_ref[...]) + b2_ref[...])
    xnew_ref[...] = jnp.dot(h, w3_ref[...]) + b3_ref[...]


def _attn_score_kernel(ev_ref, valid_ref, g_ref, qs_ref,
                       kw_ref, kb_ref, vw_ref, vb_ref,
                       a0_ref, a1_ref, vs_ref):
    ev = _gelu(ev_ref[...] * valid_ref[...])
    ks = jnp.dot(ev, kw_ref[...]) + kb_ref[...]
    vs = jnp.dot(ev, vw_ref[...]) + vb_ref[...]
    vs_ref[...] = vs
    qs_exp = jnp.dot(g_ref[...], qs_ref[...])
    prod = ks * qs_exp
    a0_ref[...] = jnp.sum(prod[:, :64], axis=1, keepdims=True) * 0.125
    a1_ref[...] = jnp.sum(prod[:, 64:], axis=1, keepdims=True) * 0.125


def _softmax_kernel(a0_ref, a1_ref, mask_ref, w0_ref, w1_ref):
    mask = mask_ref[...] > 0
    for a_ref, w_ref in ((a0_ref, w0_ref), (a1_ref, w1_ref)):
        a = jnp.where(mask, -1e9, a_ref[...])
        mx = jnp.max(a, axis=1, keepdims=True)
        e = jnp.exp(a - mx)
        w_ref[...] = e / jnp.sum(e, axis=1, keepdims=True)


def _head_kernel(w0_ref, w1_ref, vs_ref, g_ref, pe_ref, qa_ref,
                 h1pe_ref, h1pn_ref, h1qa_ref, hb1_ref,
                 h2_ref, hb2_ref, h3_ref, hb3_ref,
                 pn_ref, logits_ref):
    lane = lax.broadcasted_iota(jnp.int32, (1, 128), 1)
    wsel = jnp.where(lane < 64, w0_ref[...], w1_ref[...])
    wv = wsel * vs_ref[...]
    pooled = lax.dot_general(g_ref[...], wv, (((0,), (0,)), ((), ())))
    pn_ref[...] = pooled
    h = _gelu(jnp.dot(pe_ref[...], h1pe_ref[...])
              + jnp.dot(pooled, h1pn_ref[...])
              + jnp.dot(qa_ref[...], h1qa_ref[...]) + hb1_ref[...])
    h = _gelu(jnp.dot(h, h2_ref[...]) + hb2_ref[...])
    logits_ref[...] = jnp.dot(h, h3_ref[...]) + hb3_ref[...]


def _tw(lin):
    w, b = lin
    return w.T.astype(F32), b.reshape(1, -1).astype(F32)


def kernel(x, edge_attr, qa_attr, edge_index, node_batch, num_of_nodes, params):
    n_nodes, _ = x.shape
    n_edges = edge_index.shape[1]
    bsz = qa_attr.shape[0]
    max_node = 50
    hid = 128

    row = edge_index[0].astype(jnp.int32)
    col = edge_index[1].astype(jnp.int32)
    rowf = row.astype(F32).reshape(n_edges, 1)

    t = jnp.searchsorted(node_batch.astype(jnp.int32), jnp.arange(bsz + 1, dtype=jnp.int32))
    t = t.astype(F32)
    big = jnp.full((64 - bsz,), float(n_nodes + 10), F32)
    tlo = jnp.concatenate([t[:bsz], big]).reshape(1, 64)
    thi = jnp.concatenate([t[1:bsz + 1], big]).reshape(1, 64)

    wn_t, wn_b = _tw(params['worknode'])
    wr_t, wr_b = _tw(params['workrel'])
    gw = [_tw(l) for l in params['global_mlp']]
    e1, e2, e3 = [_tw(l) for l in params['edge_mlp']]
    w1s, w1d, w1e, w1u = (e1[0][0:128], e1[0][128:256], e1[0][256:384], e1[0][384:512])
    wm1, wm2 = [_tw(l) for l in params['weight_mlp']]
    ww1e, ww1u = wm1[0][0:128], wm1[0][128:256]
    mm1, mm2, mm3 = [_tw(l) for l in params['message_mlp']]
    m1x, m1e = mm1[0][0:128], mm1[0][128:256]
    um1, um2, um3 = [_tw(l) for l in params['update_mlp']]
    u1x, u1r, u1u = um1[0][0:128], um1[0][128:256], um1[0][256:384]
    qw_t, qw_b = _tw(params['att_q'])
    kw_t, kw_b = _tw(params['att_k'])
    vw_t, vw_b = _tw(params['att_v'])
    h1, h2, h3 = [_tw(l) for l in params['hid2out']]
    bf = jnp.bfloat16
    w1s_b, w1d_b, w1e_b = w1s.astype(bf), w1d.astype(bf), w1e.astype(bf)
    e2b, e3b = e2[0].astype(bf), e3[0].astype(bf)
    m1x_b, m1e_b = m1x.astype(bf), m1e.astype(bf)
    mm2b, mm3b = mm2[0].astype(bf), mm3[0].astype(bf)
    h1pe, h1pn, h1qa = h1[0][0:128], h1[0][128:256], h1[0][256:384]
    h3w = jnp.pad(h3[0], ((0, 0), (0, 126)))
    h3b = jnp.pad(h3[1], ((0, 0), (0, 126)))

    BE = 4000
    BN = 2000

    xp = _linproj(x.astype(F32), wn_t, wn_b, BN)
    u, u1_, u2_, u3_, qs = _prelude(
        qa_attr.astype(F32), gw, w1u, ww1u, u1u, qw_t, qw_b)
    pad14 = ((0, 64 - bsz), (0, 0))
    u1p = jnp.pad(u1_, pad14)
    u1p_b = u1p.astype(bf)
    u2p = jnp.pad(u2_, pad14)
    u3p = jnp.pad(u3_, pad14)
    qsp = jnp.pad(qs, pad14)
    nblk0 = n_edges // BE
    ea, wts, m = pl.pallas_call(
        _ea_wts_kernel,
        grid=(nblk0,),
        in_specs=[
            pl.BlockSpec((BE, 16), lambda i: (i, 0)),
            _full_spec((16, hid)), _full_spec((1, hid)),
            pl.BlockSpec((BE, 1), lambda i: (i, 0)),
            _full_spec((1, 64)), _full_spec((1, 64)),
            _full_spec((64, hid)),
            _full_spec((hid, hid)), _full_spec((1, hid)),
            _full_spec((hid, 1)), _full_spec((1, 1)),
        ],
        out_specs=[
            pl.BlockSpec((BE, hid), lambda i: (i, 0)),
            pl.BlockSpec((BE, 1), lambda i: (i, 0)),
            _full_spec((1, 64)),
        ],
        out_shape=[
            jax.ShapeDtypeStruct((n_edges, hid), F32),
            jax.ShapeDtypeStruct((n_edges, 1), F32),
            jax.ShapeDtypeStruct((1, 64), F32),
        ],
        compiler_params=pltpu.CompilerParams(
            dimension_semantics=("arbitrary",)),
    )(edge_attr.astype(F32), wr_t, wr_b, rowf, tlo, thi,
      u2p, ww1e, wm1[1], wm2[0], wm2[1])

    n_pad = _pad_rows(n_nodes)
    zeros_nd = jnp.zeros((n_pad, hid), F32)
    zeros_n16 = jnp.zeros((n_pad, 16), F32)
    ones_c16 = jnp.ones((128, 16), F32)
    cnt2 = _count_rows(col, n_nodes, ones_c16, zeros_n16)
    cnt0, cnt1 = cnt2[:n_nodes], cnt2[n_pad:n_pad + n_nodes]

    nbf = node_batch.astype(F32).reshape(n_nodes, 1)

    nblk = n_edges // BE
    sig = norm = pooled_edge = None
    for layer in range(3):
        sd = _gather_rows2(xp, row, col)

        s = pl.pallas_call(
            functools.partial(_segsum_kernel, nblk),
            grid=(nblk,),
            in_specs=[
                pl.BlockSpec((BE, 1), lambda i: (i, 0)),
                pl.BlockSpec((BE, 1), lambda i: (i, 0)),
                _full_spec((1, 64)), _full_spec((1, 64)), _full_spec((1, 64)),
            ],
            out_specs=_full_spec((1, 64)),
            out_shape=jax.ShapeDtypeStruct((1, 64), F32),
            compiler_params=pltpu.CompilerParams(
                dimension_semantics=("arbitrary",)),
        )(wts, rowf, tlo, thi, m)

        is_last = layer == 2
        in_specs = [
            pl.BlockSpec((BE, hid), lambda i: (i, 0)),
            pl.BlockSpec((BE, hid), lambda i, _n=nblk: (i + _n, 0)),
            pl.BlockSpec((BE, hid), lambda i: (i, 0)),
            pl.BlockSpec((BE, 1), lambda i: (i, 0)),
            pl.BlockSpec((BE, 1), lambda i: (i, 0)),
            _full_spec((1, 64)), _full_spec((1, 64)),
            _full_spec((1, 64)), _full_spec((1, 64)),
            _full_spec((64, hid)),
            _full_spec((hid, hid)), _full_spec((hid, hid)),
            _full_spec((hid, hid)), _full_spec((1, hid)),
            _full_spec((hid, hid)), _full_spec((1, hid)),
            _full_spec((hid, hid)), _full_spec((1, hid)),
            _full_spec((hid, hid)), _full_spec((hid, hid)),
            _full_spec((1, hid)),
            _full_spec((hid, hid)), _full_spec((1, hid)),
            _full_spec((hid, hid)), _full_spec((1, hid)),
        ]
        inputs = [sd, sd, ea, rowf, wts, tlo, thi, m, s, u1p,
                  w1s_b, w1d_b, w1e, e1[1], e2[0], e2[1], e3[0], e3[1],
                  m1x_b, m1e, mm1[1], mm2[0], mm2[1], mm3[0], mm3[1]]
        out_specs = [
            pl.BlockSpec((BE, hid), lambda i: (i, 0)),
            pl.BlockSpec((BE, hid), lambda i: (i, 0)),
        ]
        out_shape = [
            jax.ShapeDtypeStruct((n_edges, hid), F32),
            jax.ShapeDtypeStruct((n_edges, hid), F32),
        ]
        if is_last:
            out_specs += [
                pl.BlockSpec((BE, 1), lambda i: (i, 0)),
                pl.BlockSpec((BE, 1), lambda i: (i, 0)),
                _full_spec((64, hid)),
            ]
            out_shape += [
                jax.ShapeDtypeStruct((n_edges, 1), F32),
                jax.ShapeDtypeStruct((n_edges, 1), F32),
                jax.ShapeDtypeStruct((64, hid), F32),
            ]
        else:
            in_specs += [
                _full_spec((64, hid)),
                _full_spec((hid, hid)), _full_spec((1, hid)),
                _full_spec((hid, 1)), _full_spec((1, 1)),
            ]
            inputs += [u2p, ww1e, wm1[1], wm2[0], wm2[1]]
            out_specs += [
                pl.BlockSpec((BE, 1), lambda i: (i, 0)),
                _full_spec((1, 64)),
            ]
            out_shape += [
                jax.ShapeDtypeStruct((n_edges, 1), F32),
                jax.ShapeDtypeStruct((1, 64), F32),
            ]
        outs = pl.pallas_call(
            functools.partial(_edge_kernel, nblk, is_last),
            grid=(nblk,),
            in_specs=in_specs,
            out_specs=out_specs,
            out_shape=out_shape,
            compiler_params=pltpu.CompilerParams(
                dimension_semantics=("arbitrary",)),
        )(*inputs)
        if is_last:
            ne, msgs, sig, norm, pooled_edge = outs
        else:
            ne, msgs, wts, m = outs

        part = _scatter_rows(msgs, col, n_nodes, zeros_nd)

        xp = pl.pallas_call(
            _node_kernel,
            grid=(n_nodes // BN,),
            in_specs=[
                pl.BlockSpec((BN, hid), lambda i: (i, 0)),
                pl.BlockSpec((BN, hid), lambda i: (i, 0)),
                pl.BlockSpec((BN, hid), lambda i: (i, 0)),
                pl.BlockSpec((BN, 16), lambda i: (i, 0)),
                pl.BlockSpec((BN, 16), lambda i: (i, 0)),
                pl.BlockSpec((BN, 1), lambda i: (i, 0)),
                _full_spec((64, hid)),
                _full_spec((hid, hid)), _full_spec((hid, hid)),
                _full_spec((1, hid)),
                _full_spec((hid, hid)), _full_spec((1, hid)),
                _full_spec((hid, hid)), _full_spec((1, hid)),
            ],
            out_specs=pl.BlockSpec((BN, hid), lambda i: (i, 0)),
            out_shape=jax.ShapeDtypeStruct((n_nodes, hid), F32),
        )(xp, part[:n_nodes], part[n_pad:n_pad + n_nodes], cnt0, cnt1, nbf, u3p,
          u1x, u1r, um1[1], um2[0], um2[1], um3[0], um3[1])
        ea = ne

    offsets = jnp.concatenate(
        [jnp.zeros((1,), num_of_nodes.dtype), jnp.cumsum(num_of_nodes)[:-1]])
    node_pos = jnp.arange(max_node)
    gidx = jnp.clip(offsets[:, None] + node_pos[None, :], 0, n_nodes - 1)
    gidx = gidx.reshape(-1).astype(jnp.int32)
    n_ev = bsz * max_node
    n_ev_pad = 2560
    gidx_pad = jnp.concatenate(
        [gidx, jnp.zeros((n_ev_pad - n_ev,), jnp.int32)])
    valid = (node_pos[None, :] < num_of_nodes[:, None]).astype(F32)
    validf = jnp.concatenate(
        [valid.reshape(-1, 1), jnp.zeros((n_ev_pad - n_ev, 1), F32)])

    mask = node_pos[None, :] >= num_of_nodes[:, None]
    mask = mask.at[:, 0].set(jnp.where(mask.all(1), False, mask[:, 0]))
    maskf = mask.astype(F32)

    gmat = jax.nn.one_hot(
        jnp.concatenate([jnp.repeat(jnp.arange(bsz), max_node),
                         jnp.full((n_ev_pad - n_ev,), 63)]),
        64, dtype=F32)

    ev_raw = _gather_rows(xp, gidx_pad)

    a0, a1, vs = pl.pallas_call(
        _attn_score_kernel,
        grid=(1,),
        in_specs=[
            _full_spec((n_ev_pad, hid)),
            _full_spec((n_ev_pad, 1)),
            _full_spec((n_ev_pad, 64)),
            _full_spec((64, hid)),
            _full_spec((hid, hid)), _full_spec((1, hid)),
            _full_spec((hid, hid)), _full_spec((1, hid)),
        ],
        out_specs=[
            _full_spec((n_ev_pad, 1)),
            _full_spec((n_ev_pad, 1)),
            _full_spec((n_ev_pad, hid)),
        ],
        out_shape=[
            jax.ShapeDtypeStruct((n_ev_pad, 1), F32),
            jax.ShapeDtypeStruct((n_ev_pad, 1), F32),
            jax.ShapeDtypeStruct((n_ev_pad, hid), F32),
        ],
    )(ev_raw, validf, gmat, qsp, kw_t, kw_b, vw_t, vw_b)

    a0g = a0[:n_ev, 0].reshape(bsz, max_node)
    a1g = a1[:n_ev, 0].reshape(bsz, max_node)

    w0, w1 = pl.pallas_call(
        _softmax_kernel,
        grid=(1,),
        in_specs=[_full_spec((bsz, max_node))] * 3,
        out_specs=[_full_spec((bsz, max_node))] * 2,
        out_shape=[jax.ShapeDtypeStruct((bsz, max_node), F32)] * 2,
    )(a0g, a1g, maskf)

    w0f = jnp.concatenate(
        [w0.reshape(n_ev, 1), jnp.zeros((n_ev_pad - n_ev, 1), F32)])
    w1f = jnp.concatenate(
        [w1.reshape(n_ev, 1), jnp.zeros((n_ev_pad - n_ev, 1), F32)])
    qa64 = jnp.pad(qa_attr.astype(F32), ((0, 64 - bsz), (0, 0)))

    pn64, logits64 = pl.pallas_call(
        _head_kernel,
        grid=(1,),
        in_specs=[
            _full_spec((n_ev_pad, 1)), _full_spec((n_ev_pad, 1)),
            _full_spec((n_ev_pad, hid)), _full_spec((n_ev_pad, 64)),
            _full_spec((64, hid)), _full_spec((64, hid)),
            _full_spec((hid, hid)), _full_spec((hid, hid)),
            _full_spec((hid, hid)), _full_spec((1, hid)),
            _full_spec((hid, hid)), _full_spec((1, hid)),
            _full_spec((hid, hid)), _full_spec((1, hid)),
        ],
        out_specs=[_full_spec((64, hid)), _full_spec((64, hid))],
        out_shape=[jax.ShapeDtypeStruct((64, hid), F32)] * 2,
    )(w0f, w1f, vs, gmat, pooled_edge, qa64,
      h1pe, h1pn, h1qa, h1[1], h2[0], h2[1], h3w, h3b)

    logits = logits64[:bsz, :2]
    pooled_node = pn64[:bsz]
    embeddings = jnp.concatenate(
        [pooled_edge[:bsz], pooled_node, qa_attr.astype(F32)], axis=1)
    return logits, sig, norm, embeddings


# speedup vs baseline: 1.0043x; 1.0043x over previous
import functools

import jax
import jax.numpy as jnp
import numpy as np
from jax import lax
from jax.experimental import pallas as pl
from jax.experimental.pallas import tpu as pltpu
from jax.experimental.pallas import tpu_sc as plsc

F32 = jnp.float32
NEG_BIG = -1e30


_NC = 2
_NS = 16
_NW = _NC * _NS


def _sc_mesh():
    return plsc.VectorSubcoreMesh(core_axis_name="c", subcore_axis_name="s")


_K = 6


@functools.lru_cache(maxsize=None)
def _make_sc_gather(n_idx, n_rows, d, dtype_name, n_lists):
    dt = jnp.dtype(dtype_name)
    per_w = n_idx // _NW
    ch = min(128, per_w)
    n_ch = per_w // ch
    rem = per_w - n_ch * ch
    k = max(1, min(_K, n_ch))
    groups = n_ch // k
    tail = n_ch - groups * k

    scratch = [
        pltpu.VMEM((max(per_w, 8),), jnp.int32),
        pltpu.VMEM((k, ch, d), dt),
    ] + [pltpu.SemaphoreType.DMA] * (k + 1)

    @functools.partial(
        pl.kernel,
        out_type=jax.ShapeDtypeStruct((n_lists * n_idx, d), dt),
        mesh=_sc_mesh(),
        scratch_types=scratch,
    )
    def gather_k(table_hbm, *args):
        idx_hbms = args[:n_lists]
        out_hbm = args[n_lists]
        idx_v, rows_v = args[n_lists + 1:n_lists + 3]
        sems = args[n_lists + 3:]
        gsems, osem = sems[:k], sems[k]
        wid = lax.axis_index("s") * _NC + lax.axis_index("c")
        base = wid * per_w

        def run(idx_hbm, out_base):
            pltpu.sync_copy(idx_hbm.at[pl.ds(base, per_w)],
                            idx_v.at[pl.ds(0, per_w)])

            def group_body(g, carry):
                off = g * (k * ch)
                gd = [
                    pltpu.async_copy(
                        table_hbm.at[idx_v.at[pl.ds(off + b * ch, ch)]],
                        rows_v.at[b], gsems[b])
                    for b in range(k)
                ]
                od = []
                for b in range(k):
                    gd[b].wait()
                    od.append(pltpu.async_copy(
                        rows_v.at[b],
                        out_hbm.at[pl.ds(out_base + off + b * ch, ch)], osem))
                for dsc in od:
                    dsc.wait()
                return carry

            if groups:
                lax.fori_loop(0, groups, group_body, 0)
            for j in range(tail):
                off = (groups * k + j) * ch
                pltpu.async_copy(
                    table_hbm.at[idx_v.at[pl.ds(off, ch)]],
                    rows_v.at[0], gsems[0]).wait()
                pltpu.sync_copy(rows_v.at[0],
                                out_hbm.at[pl.ds(out_base + off, ch)])
            if rem:
                off = n_ch * ch
                pltpu.async_copy(
                    table_hbm.at[idx_v.at[pl.ds(off, rem)]],
                    rows_v.at[0].at[pl.ds(0, rem)], gsems[0]).wait()
                pltpu.sync_copy(rows_v.at[0].at[pl.ds(0, rem)],
                                out_hbm.at[pl.ds(out_base + off, rem)])

        for j, idx_hbm in enumerate(idx_hbms):
            run(idx_hbm, j * n_idx + base)

    return gather_k


def _pad_rows(n_rows):
    unit = 8 * _NS
    return ((n_rows + unit - 1) // unit) * unit


@functools.lru_cache(maxsize=None)
def _make_sc_scatter(n_idx, n_rows, d, with_count):
    per_w = n_idx // _NW
    ch = min(64, per_w)
    n_ch = per_w // ch
    rem = per_w - n_ch * ch
    k = max(1, min(3 if with_count else 4, n_ch))
    groups = n_ch // k
    tail = n_ch - groups * k
    n_pad = _pad_rows(n_rows)
    rows_per_tile = n_pad // _NS

    scratch = (
        [pltpu.VMEM((ch,), jnp.int32)] * k
        + [pltpu.VMEM((ch, d), F32)] * k
        + [pltpu.VMEM_SHARED((n_pad, d), F32)]
        + [pltpu.SemaphoreType.DMA] * (2 * k + 1)
    )
    out_type = [jax.ShapeDtypeStruct((2 * n_pad, d), F32)]
    if with_count:
        scratch = scratch + [pltpu.VMEM((ch, 16), F32),
                             pltpu.VMEM_SHARED((n_pad, 16), F32)]
        out_type = out_type + [jax.ShapeDtypeStruct((2 * n_pad, 16), F32)]

    @functools.partial(
        pl.kernel,
        out_type=out_type,
        mesh=_sc_mesh(),
        scratch_types=scratch,
    )
    def scatter_k(*args):
        if with_count:
            (vals_hbm, idx_hbm, zeros_hbm, ones_hbm, zeros16_hbm,
             out_hbm, cnt_hbm) = args[:7]
            rest = args[7:]
        else:
            vals_hbm, idx_hbm, zeros_hbm, out_hbm = args[:4]
            rest = args[4:]
        idx_vs = rest[:k]
        rows_vs = rest[k:2 * k]
        acc_sh = rest[2 * k]
        isems = rest[2 * k + 1:3 * k + 1]
        msems = rest[3 * k + 1:4 * k + 1]
        ssem = rest[4 * k + 1]
        if with_count:
            ones_v, acc16_sh = rest[4 * k + 2:4 * k + 4]
        c = lax.axis_index("c")
        s = lax.axis_index("s")
        wid = s * _NC + c
        base = wid * per_w
        pltpu.sync_copy(
            zeros_hbm.at[pl.ds(s * rows_per_tile, rows_per_tile)],
            acc_sh.at[pl.ds(s * rows_per_tile, rows_per_tile)],
        )
        if with_count:
            pltpu.sync_copy(ones_hbm, ones_v)
            pltpu.sync_copy(
                zeros16_hbm.at[pl.ds(s * rows_per_tile, rows_per_tile)],
                acc16_sh.at[pl.ds(s * rows_per_tile, rows_per_tile)],
            )
        plsc.subcore_barrier()

        def group_body(g, carry):
            off = base + g * (k * ch)
            idd = [pltpu.async_copy(idx_hbm.at[pl.ds(off + b * ch, ch)],
                                    idx_vs[b], isems[b]) for b in range(k)]
            mdd = [pltpu.async_copy(vals_hbm.at[pl.ds(off + b * ch, ch)],
                                    rows_vs[b], msems[b]) for b in range(k)]
            sd = []
            for b in range(k):
                idd[b].wait()
                mdd[b].wait()
                sd.append(pltpu.async_copy(
                    rows_vs[b], acc_sh.at[idx_vs[b]], ssem, add=True))
                if with_count:
                    sd.append(pltpu.async_copy(
                        ones_v, acc16_sh.at[idx_vs[b]], ssem, add=True))
            for dsc in sd:
                dsc.wait()
            return carry

        if groups:
            lax.fori_loop(0, groups, group_body, 0)

        def do_serial(off, cnt):
            pltpu.sync_copy(idx_hbm.at[pl.ds(off, cnt)],
                            idx_vs[0].at[pl.ds(0, cnt)])
            pltpu.sync_copy(vals_hbm.at[pl.ds(off, cnt)],
                            rows_vs[0].at[pl.ds(0, cnt)])
            pltpu.sync_copy(
                rows_vs[0].at[pl.ds(0, cnt)],
                acc_sh.at[idx_vs[0].at[pl.ds(0, cnt)]],
                add=True,
            )
            if with_count:
                pltpu.sync_copy(
                    ones_v.at[pl.ds(0, cnt)],
                    acc16_sh.at[idx_vs[0].at[pl.ds(0, cnt)]],
                    add=True,
                )

        for j in range(tail):
            do_serial(base + (groups * k + j) * ch, ch)
        if rem:
            do_serial(base + n_ch * ch, rem)
        plsc.subcore_barrier()
        pltpu.sync_copy(
            acc_sh.at[pl.ds(s * rows_per_tile, rows_per_tile)],
            out_hbm.at[pl.ds(c * n_pad + s * rows_per_tile, rows_per_tile)],
        )
        if with_count:
            pltpu.sync_copy(
                acc16_sh.at[pl.ds(s * rows_per_tile, rows_per_tile)],
                cnt_hbm.at[pl.ds(c * n_pad + s * rows_per_tile, rows_per_tile)],
            )

    return scatter_k


@functools.lru_cache(maxsize=None)
def _make_sc_count(n_idx, n_rows, d):
    per_w = n_idx // _NW
    ch = min(128, per_w)
    n_ch = per_w // ch
    rem = per_w - n_ch * ch
    k = max(1, min(_K, n_ch))
    groups = n_ch // k
    tail = n_ch - groups * k
    n_pad = _pad_rows(n_rows)
    rows_per_tile = n_pad // _NS

    scratch = (
        [pltpu.VMEM((ch,), jnp.int32)] * k
        + [pltpu.VMEM((ch, d), F32)]
        + [pltpu.VMEM_SHARED((n_pad, d), F32)]
        + [pltpu.SemaphoreType.DMA] * (k + 1)
    )

    @functools.partial(
        pl.kernel,
        out_type=jax.ShapeDtypeStruct((2 * n_pad, d), F32),
        mesh=_sc_mesh(),
        scratch_types=scratch,
    )
    def count_k(idx_hbm, ones_hbm, zeros_hbm, out_hbm, *rest):
        idx_vs = rest[:k]
        ones_v = rest[k]
        acc_sh = rest[k + 1]
        isems = rest[k + 2:2 * k + 2]
        ssem = rest[2 * k + 2]
        c = lax.axis_index("c")
        s = lax.axis_index("s")
        wid = s * _NC + c
        base = wid * per_w
        pltpu.sync_copy(ones_hbm, ones_v)
        pltpu.sync_copy(
            zeros_hbm.at[pl.ds(s * rows_per_tile, rows_per_tile)],
            acc_sh.at[pl.ds(s * rows_per_tile, rows_per_tile)],
        )
        plsc.subcore_barrier()

        def group_body(g, carry):
            off = base + g * (k * ch)
            idd = [pltpu.async_copy(idx_hbm.at[pl.ds(off + b * ch, ch)],
                                    idx_vs[b], isems[b]) for b in range(k)]
            sd = []
            for b in range(k):
                idd[b].wait()
                sd.append(pltpu.async_copy(
                    ones_v, acc_sh.at[idx_vs[b]], ssem, add=True))
            for dsc in sd:
                dsc.wait()
            return carry

        if groups:
            lax.fori_loop(0, groups, group_body, 0)

        def do_serial(off, cnt):
            pltpu.sync_copy(idx_hbm.at[pl.ds(off, cnt)],
                            idx_vs[0].at[pl.ds(0, cnt)])
            pltpu.sync_copy(
                ones_v.at[pl.ds(0, cnt)],
                acc_sh.at[idx_vs[0].at[pl.ds(0, cnt)]],
                add=True,
            )

        for j in range(tail):
            do_serial(base + (groups * k + j) * ch, ch)
        if rem:
            do_serial(base + n_ch * ch, rem)
        plsc.subcore_barrier()
        pltpu.sync_copy(
            acc_sh.at[pl.ds(s * rows_per_tile, rows_per_tile)],
            out_hbm.at[pl.ds(c * n_pad + s * rows_per_tile, rows_per_tile)],
        )

    return count_k


def _count_rows(idx, n_rows, ones, zeros):
    return _make_sc_count(idx.shape[0], n_rows, ones.shape[1])(
        idx, ones, zeros)


def _gather_rows(table, idx):
    return _make_sc_gather(idx.shape[0], table.shape[0], table.shape[1],
                           str(table.dtype), 1)(table, idx)


def _gather_rows2(table, idx_a, idx_b):
    return _make_sc_gather(idx_a.shape[0], table.shape[0], table.shape[1],
                           str(table.dtype), 2)(table, idx_a, idx_b)


def _scatter_rows(vals, idx, n_rows, zeros):
    out = _make_sc_scatter(vals.shape[0], n_rows, vals.shape[1], False)(
        vals, idx, zeros)
    return out[0]


def _scatter_rows_count(vals, idx, n_rows, zeros, ones16, zeros16):
    return _make_sc_scatter(vals.shape[0], n_rows, vals.shape[1], True)(
        vals, idx, zeros, ones16, zeros16)


def _gelu(v):
    return jax.nn.gelu(v)


def _full_spec(shape):
    nd = len(shape)
    return pl.BlockSpec(shape, lambda i, _nd=nd: (0,) * _nd)


def _proj_kernel(x_ref, w_ref, b_ref, o_ref):
    o_ref[...] = jnp.dot(x_ref[...], w_ref[...]) + b_ref[...]


def _linproj(xin, wt, b, blk):
    n, din = xin.shape
    dout = wt.shape[1]
    grid = n // blk
    return pl.pallas_call(
        _proj_kernel,
        grid=(grid,),
        in_specs=[
            pl.BlockSpec((blk, din), lambda i: (i, 0)),
            _full_spec((din, dout)),
            _full_spec((1, dout)),
        ],
        out_specs=pl.BlockSpec((blk, dout), lambda i: (i, 0)),
        out_shape=jax.ShapeDtypeStruct((n, dout), F32),
    )(xin, wt, b)


def _prelude_kernel(qa_ref, g1_ref, gb1_ref, g2_ref, gb2_ref, g3_ref, gb3_ref,
                    w1ue_ref, w1uw_ref, w1uu_ref, qw_ref, qb_ref,
                    u_ref, u1_ref, u2_ref, u3_ref, qs_ref):
    qa = qa_ref[...]
    h = _gelu(jnp.dot(qa, g1_ref[...]) + gb1_ref[...])
    h = _gelu(jnp.dot(h, g2_ref[...]) + gb2_ref[...])
    u = jnp.dot(h, g3_ref[...]) + gb3_ref[...]
    u_ref[...] = u
    u1_ref[...] = jnp.dot(u, w1ue_ref[...])
    u2_ref[...] = jnp.dot(u, w1uw_ref[...])
    u3_ref[...] = jnp.dot(u, w1uu_ref[...])
    qs_ref[...] = jnp.dot(qa, qw_ref[...]) + qb_ref[...]


def _prelude(qa, gw, w1ue, w1uw, w1uu, qw, qb):
    bsz, dfeat = qa.shape
    hid = gw[0][0].shape[1]
    outs = [jax.ShapeDtypeStruct((bsz, hid), F32)] * 5
    specs = [_full_spec((bsz, dfeat))]
    for (wt, b) in gw:
        specs.append(_full_spec(wt.shape))
        specs.append(_full_spec(b.shape))
    specs += [_full_spec(w1ue.shape), _full_spec(w1uw.shape),
              _full_spec(w1uu.shape), _full_spec(qw.shape), _full_spec(qb.shape)]
    return pl.pallas_call(
        _prelude_kernel,
        grid=(1,),
        in_specs=specs,
        out_specs=[_full_spec((bsz, hid))] * 5,
        out_shape=outs,
    )(qa, gw[0][0], gw[0][1], gw[1][0], gw[1][1], gw[2][0], gw[2][1],
      w1ue, w1uw, w1uu, qw, qb)


def _onehot_from_bounds(rowf, tlo, thi):
    ge = rowf >= tlo
    lt = rowf < thi
    return jnp.where(ge & lt, 1.0, 0.0).astype(F32)


def _wts_and_max(ea, oh, pid, u2_ref, w1_ref, b1_ref, w2_ref, b2_ref,
                 wts_ref, m_ref):
    @pl.when(pid == 0)
    def _():
        m_ref[...] = jnp.full(m_ref.shape, NEG_BIG, F32)

    h = _gelu(jnp.dot(ea, w1_ref[...]) + jnp.dot(oh, u2_ref[...])
              + b1_ref[...])
    wts = jnp.dot(h, w2_ref[...]) + b2_ref[...]
    wts_ref[...] = wts
    contrib = jnp.where(oh > 0, wts, NEG_BIG)
    m_ref[...] = jnp.maximum(m_ref[...],
                             jnp.max(contrib, axis=0, keepdims=True))


def _ea_wts_kernel(ea16_ref, wr_ref, wrb_ref, rowf_ref, tlo_ref, thi_ref,
                   u2_ref, w1_ref, b1_ref, w2_ref, b2_ref,
                   ea_ref, wts_ref, m_ref):
    pid = pl.program_id(0)
    ea = jnp.dot(ea16_ref[...], wr_ref[...]) + wrb_ref[...]
    ea_ref[...] = ea
    oh = _onehot_from_bounds(rowf_ref[...], tlo_ref[...], thi_ref[...])
    _wts_and_max(ea, oh, pid, u2_ref, w1_ref, b1_ref, w2_ref, b2_ref,
                 wts_ref, m_ref)


def _segsum_kernel(nblk, wts_ref, rowf_ref, tlo_ref, thi_ref, m_ref, s_ref):
    pid = pl.program_id(0)

    @pl.when(pid == 0)
    def _():
        s_ref[...] = jnp.zeros(s_ref.shape, F32)

    oh = _onehot_from_bounds(rowf_ref[...], tlo_ref[...], thi_ref[...])
    m_e = jnp.sum(oh * m_ref[...], axis=1, keepdims=True)
    e = jnp.exp(wts_ref[...] - m_e)
    s_ref[...] = s_ref[...] + jnp.sum(oh * e, axis=0, keepdims=True)


def _edge_kernel(nblk, is_last,
                 src_ref, dest_ref, ea_ref, rowf_ref, wts_ref,
                 tlo_ref, thi_ref, m_ref, s_ref, u1_ref,
                 w1s_ref, w1d_ref, w1e_ref, b1_ref, w2_ref, b2_ref,
                 w3_ref, b3_ref,
                 m1x_ref, m1e_ref, mb1_ref, m2_ref, mb2_ref, m3_ref, mb3_ref,
                 *refs):
    if is_last:
        ne_ref, msgs_ref, sig_ref, norm_ref, pooled_ref = refs
    else:
        (u2_ref, ww1_ref, wwb1_ref, ww2_ref, wwb2_ref,
         ne_ref, msgs_ref, wtsn_ref, mn_ref) = refs
    pid = pl.program_id(0)
    oh = _onehot_from_bounds(rowf_ref[...], tlo_ref[...], thi_ref[...])
    wts = wts_ref[...]
    m_e = jnp.sum(oh * m_ref[...], axis=1, keepdims=True)
    s_e = jnp.sum(oh * s_ref[...], axis=1, keepdims=True)
    norm = jnp.exp(wts - m_e) / jnp.maximum(s_e, 1e-16)
    src_b = src_ref[...].astype(jnp.bfloat16)
    dest_b = dest_ref[...].astype(jnp.bfloat16)
    h = _gelu(jnp.dot(src_b, w1s_ref[...], preferred_element_type=F32)
              + jnp.dot(dest_b, w1d_ref[...], preferred_element_type=F32)
              + jnp.dot(ea_ref[...], w1e_ref[...]) + jnp.dot(oh, u1_ref[...])
              + b1_ref[...])
    h = _gelu(jnp.dot(h, w2_ref[...]) + b2_ref[...])
    ne = jnp.dot(h, w3_ref[...]) + b3_ref[...]
    ne_ref[...] = ne
    g = _gelu(jnp.dot(src_b, m1x_ref[...], preferred_element_type=F32)
              + jnp.dot(ne, m1e_ref[...]) + mb1_ref[...])
    g = _gelu(jnp.dot(g, m2_ref[...]) + mb2_ref[...])
    msg = jnp.dot(g, m3_ref[...]) + mb3_ref[...]
    msgs_ref[...] = msg * norm
    if is_last:
        sig_ref[...] = jax.nn.sigmoid(wts)
        norm_ref[...] = norm

        @pl.when(pid == 0)
        def _():
            pooled_ref[...] = jnp.zeros(pooled_ref.shape, F32)

        pooled_ref[...] = pooled_ref[...] + lax.dot_general(
            oh, ne * norm, (((0,), (0,)), ((), ())))
    else:
        _wts_and_max(ne, oh, pid, u2_ref, ww1_ref, wwb1_ref, ww2_ref,
                     wwb2_ref, wtsn_ref, mn_ref)


def _node_kernel(xold_ref, p0_ref, p1_ref, c0_ref, c1_ref, nbf_ref, u3_ref,
                 w1x_ref, w1r_ref, b1_ref, w2_ref, b2_ref, w3_ref, b3_ref,
                 xnew_ref):
    cnt = c0_ref[...][:, :1] + c1_ref[...][:, :1]
    received = (p0_ref[...] + p1_ref[...]) / jnp.maximum(cnt, 1.0)
    seg = lax.broadcasted_iota(jnp.int32, (1, 64), 1).astype(F32)
    ohn = jnp.where(nbf_ref[...] == seg, 1.0, 0.0).astype(F32)
    h = _gelu(jnp.dot(xold_ref[...], w1x_ref[...])
              + jnp.dot(received, w1r_ref[...])
              + jnp.dot(ohn, u3_ref[...]) + b1_ref[...])
    h = _gelu(jnp.dot(h, w2_ref[...]) + b2_ref[...])
    xnew_ref[...] = jnp.dot(h, w3_ref[...]) + b3_ref[...]


def _attn_score_kernel(ev_ref, valid_ref, g_ref, qs_ref,
                       kw_ref, kb_ref, vw_ref, vb_ref,
                       a0_ref, a1_ref, vs_ref):
    ev = _gelu(ev_ref[...] * valid_ref[...])
    ks = jnp.dot(ev, kw_ref[...]) + kb_ref[...]
    vs = jnp.dot(ev, vw_ref[...]) + vb_ref[...]
    vs_ref[...] = vs
    qs_exp = jnp.dot(g_ref[...], qs_ref[...])
    prod = ks * qs_exp
    a0_ref[...] = jnp.sum(prod[:, :64], axis=1, keepdims=True) * 0.125
    a1_ref[...] = jnp.sum(prod[:, 64:], axis=1, keepdims=True) * 0.125


def _softmax_kernel(a0_ref, a1_ref, mask_ref, w0_ref, w1_ref):
    mask = mask_ref[...] > 0
    for a_ref, w_ref in ((a0_ref, w0_ref), (a1_ref, w1_ref)):
        a = jnp.where(mask, -1e9, a_ref[...])
        mx = jnp.max(a, axis=1, keepdims=True)
        e = jnp.exp(a - mx)
        w_ref[...] = e / jnp.sum(e, axis=1, keepdims=True)


def _head_kernel(w0_ref, w1_ref, vs_ref, g_ref, pe_ref, qa_ref,
                 h1pe_ref, h1pn_ref, h1qa_ref, hb1_ref,
                 h2_ref, hb2_ref, h3_ref, hb3_ref,
                 pn_ref, logits_ref):
    lane = lax.broadcasted_iota(jnp.int32, (1, 128), 1)
    wsel = jnp.where(lane < 64, w0_ref[...], w1_ref[...])
    wv = wsel * vs_ref[...]
    pooled = lax.dot_general(g_ref[...], wv, (((0,), (0,)), ((), ())))
    pn_ref[...] = pooled
    h = _gelu(jnp.dot(pe_ref[...], h1pe_ref[...])
              + jnp.dot(pooled, h1pn_ref[...])
              + jnp.dot(qa_ref[...], h1qa_ref[...]) + hb1_ref[...])
    h = _gelu(jnp.dot(h, h2_ref[...]) + hb2_ref[...])
    logits_ref[...] = jnp.dot(h, h3_ref[...]) + hb3_ref[...]


def _tw(lin):
    w, b = lin
    return w.T.astype(F32), b.reshape(1, -1).astype(F32)


def kernel(x, edge_attr, qa_attr, edge_index, node_batch, num_of_nodes, params):
    n_nodes, _ = x.shape
    n_edges = edge_index.shape[1]
    bsz = qa_attr.shape[0]
    max_node = 50
    hid = 128

    row = edge_index[0].astype(jnp.int32)
    col = edge_index[1].astype(jnp.int32)
    rowf = row.astype(F32).reshape(n_edges, 1)

    t = jnp.searchsorted(node_batch.astype(jnp.int32), jnp.arange(bsz + 1, dtype=jnp.int32))
    t = t.astype(F32)
    big = jnp.full((64 - bsz,), float(n_nodes + 10), F32)
    tlo = jnp.concatenate([t[:bsz], big]).reshape(1, 64)
    thi = jnp.concatenate([t[1:bsz + 1], big]).reshape(1, 64)

    wn_t, wn_b = _tw(params['worknode'])
    wr_t, wr_b = _tw(params['workrel'])
    gw = [_tw(l) for l in params['global_mlp']]
    e1, e2, e3 = [_tw(l) for l in params['edge_mlp']]
    w1s, w1d, w1e, w1u = (e1[0][0:128], e1[0][128:256], e1[0][256:384], e1[0][384:512])
    wm1, wm2 = [_tw(l) for l in params['weight_mlp']]
    ww1e, ww1u = wm1[0][0:128], wm1[0][128:256]
    mm1, mm2, mm3 = [_tw(l) for l in params['message_mlp']]
    m1x, m1e = mm1[0][0:128], mm1[0][128:256]
    um1, um2, um3 = [_tw(l) for l in params['update_mlp']]
    u1x, u1r, u1u = um1[0][0:128], um1[0][128:256], um1[0][256:384]
    qw_t, qw_b = _tw(params['att_q'])
    kw_t, kw_b = _tw(params['att_k'])
    vw_t, vw_b = _tw(params['att_v'])
    h1, h2, h3 = [_tw(l) for l in params['hid2out']]
    bf = jnp.bfloat16
    w1s_b, w1d_b, w1e_b = w1s.astype(bf), w1d.astype(bf), w1e.astype(bf)
    e2b, e3b = e2[0].astype(bf), e3[0].astype(bf)
    m1x_b, m1e_b = m1x.astype(bf), m1e.astype(bf)
    mm2b, mm3b = mm2[0].astype(bf), mm3[0].astype(bf)
    h1pe, h1pn, h1qa = h1[0][0:128], h1[0][128:256], h1[0][256:384]
    h3w = jnp.pad(h3[0], ((0, 0), (0, 126)))
    h3b = jnp.pad(h3[1], ((0, 0), (0, 126)))

    BE = 4000
    BN = 2000

    xp = _linproj(x.astype(F32), wn_t, wn_b, BN)
    u, u1_, u2_, u3_, qs = _prelude(
        qa_attr.astype(F32), gw, w1u, ww1u, u1u, qw_t, qw_b)
    pad14 = ((0, 64 - bsz), (0, 0))
    u1p = jnp.pad(u1_, pad14)
    u1p_b = u1p.astype(bf)
    u2p = jnp.pad(u2_, pad14)
    u3p = jnp.pad(u3_, pad14)
    qsp = jnp.pad(qs, pad14)
    nblk0 = n_edges // BE
    ea, wts, m = pl.pallas_call(
        _ea_wts_kernel,
        grid=(nblk0,),
        in_specs=[
            pl.BlockSpec((BE, 16), lambda i: (i, 0)),
            _full_spec((16, hid)), _full_spec((1, hid)),
            pl.BlockSpec((BE, 1), lambda i: (i, 0)),
            _full_spec((1, 64)), _full_spec((1, 64)),
            _full_spec((64, hid)),
            _full_spec((hid, hid)), _full_spec((1, hid)),
            _full_spec((hid, 1)), _full_spec((1, 1)),
        ],
        out_specs=[
            pl.BlockSpec((BE, hid), lambda i: (i, 0)),
            pl.BlockSpec((BE, 1), lambda i: (i, 0)),
            _full_spec((1, 64)),
        ],
        out_shape=[
            jax.ShapeDtypeStruct((n_edges, hid), F32),
            jax.ShapeDtypeStruct((n_edges, 1), F32),
            jax.ShapeDtypeStruct((1, 64), F32),
        ],
        compiler_params=pltpu.CompilerParams(
            dimension_semantics=("arbitrary",)),
    )(edge_attr.astype(F32), wr_t, wr_b, rowf, tlo, thi,
      u2p, ww1e, wm1[1], wm2[0], wm2[1])

    n_pad = _pad_rows(n_nodes)
    zeros_nd = jnp.zeros((n_pad, hid), F32)
    zeros_n16 = jnp.zeros((n_pad, 16), F32)
    ones_c16 = jnp.ones((128, 16), F32)
    cnt2 = _count_rows(col, n_nodes, ones_c16, zeros_n16)
    cnt0, cnt1 = cnt2[:n_nodes], cnt2[n_pad:n_pad + n_nodes]

    nbf = node_batch.astype(F32).reshape(n_nodes, 1)

    nblk = n_edges // BE
    sig = norm = pooled_edge = None
    for layer in range(3):
        sd = _gather_rows2(xp, row, col)

        s = pl.pallas_call(
            functools.partial(_segsum_kernel, nblk),
            grid=(nblk,),
            in_specs=[
                pl.BlockSpec((BE, 1), lambda i: (i, 0)),
                pl.BlockSpec((BE, 1), lambda i: (i, 0)),
                _full_spec((1, 64)), _full_spec((1, 64)), _full_spec((1, 64)),
            ],
            out_specs=_full_spec((1, 64)),
            out_shape=jax.ShapeDtypeStruct((1, 64), F32),
            compiler_params=pltpu.CompilerParams(
                dimension_semantics=("arbitrary",)),
        )(wts, rowf, tlo, thi, m)

        is_last = layer == 2
        in_specs = [
            pl.BlockSpec((BE, hid), lambda i: (i, 0)),
            pl.BlockSpec((BE, hid), lambda i, _n=nblk: (i + _n, 0)),
            pl.BlockSpec((BE, hid), lambda i: (i, 0)),
            pl.BlockSpec((BE, 1), lambda i: (i, 0)),
            pl.BlockSpec((BE, 1), lambda i: (i, 0)),
            _full_spec((1, 64)), _full_spec((1, 64)),
            _full_spec((1, 64)), _full_spec((1, 64)),
            _full_spec((64, hid)),
            _full_spec((hid, hid)), _full_spec((hid, hid)),
            _full_spec((hid, hid)), _full_spec((1, hid)),
            _full_spec((hid, hid)), _full_spec((1, hid)),
            _full_spec((hid, hid)), _full_spec((1, hid)),
            _full_spec((hid, hid)), _full_spec((hid, hid)),
            _full_spec((1, hid)),
            _full_spec((hid, hid)), _full_spec((1, hid)),
            _full_spec((hid, hid)), _full_spec((1, hid)),
        ]
        inputs = [sd, sd, ea, rowf, wts, tlo, thi, m, s, u1p,
                  w1s_b, w1d_b, w1e, e1[1], e2[0], e2[1], e3[0], e3[1],
                  m1x_b, m1e, mm1[1], mm2[0], mm2[1], mm3[0], mm3[1]]
        out_specs = [
            pl.BlockSpec((BE, hid), lambda i: (i, 0)),
            pl.BlockSpec((BE, hid), lambda i: (i, 0)),
        ]
        out_shape = [
            jax.ShapeDtypeStruct((n_edges, hid), F32),
            jax.ShapeDtypeStruct((n_edges, hid), F32),
        ]
        if is_last:
            out_specs += [
                pl.BlockSpec((BE, 1), lambda i: (i, 0)),
                pl.BlockSpec((BE, 1), lambda i: (i, 0)),
                _full_spec((64, hid)),
            ]
            out_shape += [
                jax.ShapeDtypeStruct((n_edges, 1), F32),
                jax.ShapeDtypeStruct((n_edges, 1), F32),
                jax.ShapeDtypeStruct((64, hid), F32),
            ]
        else:
            in_specs += [
                _full_spec((64, hid)),
                _full_spec((hid, hid)), _full_spec((1, hid)),
                _full_spec((hid, 1)), _full_spec((1, 1)),
            ]
            inputs += [u2p, ww1e, wm1[1], wm2[0], wm2[1]]
            out_specs += [
                pl.BlockSpec((BE, 1), lambda i: (i, 0)),
                _full_spec((1, 64)),
            ]
            out_shape += [
                jax.ShapeDtypeStruct((n_edges, 1), F32),
                jax.ShapeDtypeStruct((1, 64), F32),
            ]
        outs = pl.pallas_call(
            functools.partial(_edge_kernel, nblk, is_last),
            grid=(nblk,),
            in_specs=in_specs,
            out_specs=out_specs,
            out_shape=out_shape,
            compiler_params=pltpu.CompilerParams(
                dimension_semantics=("arbitrary",)),
        )(*inputs)
        if is_last:
            ne, msgs, sig, norm, pooled_edge = outs
        else:
            ne, msgs, wts, m = outs

        part = _scatter_rows(msgs, col, n_nodes, zeros_nd)

        xp = pl.pallas_call(
            _node_kernel,
            grid=(n_nodes // BN,),
            in_specs=[
                pl.BlockSpec((BN, hid), lambda i: (i, 0)),
                pl.BlockSpec((BN, hid), lambda i: (i, 0)),
                pl.BlockSpec((BN, hid), lambda i: (i, 0)),
                pl.BlockSpec((BN, 16), lambda i: (i, 0)),
                pl.BlockSpec((BN, 16), lambda i: (i, 0)),
                pl.BlockSpec((BN, 1), lambda i: (i, 0)),
                _full_spec((64, hid)),
                _full_spec((hid, hid)), _full_spec((hid, hid)),
                _full_spec((1, hid)),
                _full_spec((hid, hid)), _full_spec((1, hid)),
                _full_spec((hid, hid)), _full_spec((1, hid)),
            ],
            out_specs=pl.BlockSpec((BN, hid), lambda i: (i, 0)),
            out_shape=jax.ShapeDtypeStruct((n_nodes, hid), F32),
        )(xp, part[:n_nodes], part[n_pad:n_pad + n_nodes], cnt0, cnt1, nbf, u3p,
          u1x, u1r, um1[1], um2[0], um2[1], um3[0], um3[1])
        ea = ne

    offsets = jnp.concatenate(
        [jnp.zeros((1,), num_of_nodes.dtype), jnp.cumsum(num_of_nodes)[:-1]])
    node_pos = jnp.arange(max_node)
    gidx = jnp.clip(offsets[:, None] + node_pos[None, :], 0, n_nodes - 1)
    gidx = gidx.reshape(-1).astype(jnp.int32)
    n_ev = bsz * max_node
    n_ev_pad = 2560
    gidx_pad = jnp.concatenate(
        [gidx, jnp.zeros((n_ev_pad - n_ev,), jnp.int32)])
    valid = (node_pos[None, :] < num_of_nodes[:, None]).astype(F32)
    validf = jnp.concatenate(
        [valid.reshape(-1, 1), jnp.zeros((n_ev_pad - n_ev, 1), F32)])

    mask = node_pos[None, :] >= num_of_nodes[:, None]
    mask = mask.at[:, 0].set(jnp.where(mask.all(1), False, mask[:, 0]))
    maskf = mask.astype(F32)

    gmat = jax.nn.one_hot(
        jnp.concatenate([jnp.repeat(jnp.arange(bsz), max_node),
                         jnp.full((n_ev_pad - n_ev,), 63)]),
        64, dtype=F32)

    ev_raw = _gather_rows(xp, gidx_pad)

    a0, a1, vs = pl.pallas_call(
        _attn_score_kernel,
        grid=(1,),
        in_specs=[
            _full_spec((n_ev_pad, hid)),
            _full_spec((n_ev_pad, 1)),
            _full_spec((n_ev_pad, 64)),
            _full_spec((64, hid)),
            _full_spec((hid, hid)), _full_spec((1, hid)),
            _full_spec((hid, hid)), _full_spec((1, hid)),
        ],
        out_specs=[
            _full_spec((n_ev_pad, 1)),
            _full_spec((n_ev_pad, 1)),
            _full_spec((n_ev_pad, hid)),
        ],
        out_shape=[
            jax.ShapeDtypeStruct((n_ev_pad, 1), F32),
            jax.ShapeDtypeStruct((n_ev_pad, 1), F32),
            jax.ShapeDtypeStruct((n_ev_pad, hid), F32),
        ],
    )(ev_raw, validf, gmat, qsp, kw_t, kw_b, vw_t, vw_b)

    a0g = a0[:n_ev, 0].reshape(bsz, max_node)
    a1g = a1[:n_ev, 0].reshape(bsz, max_node)

    w0, w1 = pl.pallas_call(
        _softmax_kernel,
        grid=(1,),
        in_specs=[_full_spec((bsz, max_node))] * 3,
        out_specs=[_full_spec((bsz, max_node))] * 2,
        out_shape=[jax.ShapeDtypeStruct((bsz, max_node), F32)] * 2,
    )(a0g, a1g, maskf)

    w0f = jnp.concatenate(
        [w0.reshape(n_ev, 1), jnp.zeros((n_ev_pad - n_ev, 1), F32)])
    w1f = jnp.concatenate(
        [w1.reshape(n_ev, 1), jnp.zeros((n_ev_pad - n_ev, 1), F32)])
    qa64 = jnp.pad(qa_attr.astype(F32), ((0, 64 - bsz), (0, 0)))

    pn64, logits64 = pl.pallas_call(
        _head_kernel,
        grid=(1,),
        in_specs=[
            _full_spec((n_ev_pad, 1)), _full_spec((n_ev_pad, 1)),
            _full_spec((n_ev_pad, hid)), _full_spec((n_ev_pad, 64)),
            _full_spec((64, hid)), _full_spec((64, hid)),
            _full_spec((hid, hid)), _full_spec((hid, hid)),
            _full_spec((hid, hid)), _full_spec((1, hid)),
            _full_spec((hid, hid)), _full_spec((1, hid)),
            _full_spec((hid, hid)), _full_spec((1, hid)),
        ],
        out_specs=[_full_spec((64, hid)), _full_spec((64, hid))],
        out_shape=[jax.ShapeDtypeStruct((64, hid), F32)] * 2,
    )(w0f, w1f, vs, gmat, pooled_edge, qa64,
      h1pe, h1pn, h1qa, h1[1], h2[0], h2[1], h3w, h3b)

    logits = logits64[:bsz, :2]
    pooled_node = pn64[:bsz]
    embeddings = jnp.concatenate(
        [pooled_edge[:bsz], pooled_node, qa_attr.astype(F32)], axis=1)
    return logits, sig, norm, embeddings
